# baseline (speedup 1.0000x reference)
import dataclasses
import functools

import jax
import jax.numpy as jnp
from jax import lax
from jax.experimental import pallas as pl
from jax.experimental.pallas import tpu as pltpu
from jax.experimental.pallas import tpu_sc as plsc

N_NODES = 10000
N_EDGES = 160000
IN_FEAT = 256
OUT_FEAT = 256
NUM_RELS = 8

NC = 2
NS = 16
LANES = 16
NWK = NC * NS

BR = 320
ACC_R = BR + 1
CAP = 8192
CHUNK = 128
ABLK = 4000
F_V = OUT_FEAT // LANES


def _sc_compiler_params():
    cp = pltpu.CompilerParams()
    if "needs_layout_passes" in pltpu.CompilerParams.__dataclass_fields__:
        cp = dataclasses.replace(cp, needs_layout_passes=False)
    return cp


def _xw_body(x_ref, w_ref, o_ref):
    xb = jnp.dot(x_ref[...].astype(jnp.bfloat16),
                 w_ref[0].astype(jnp.bfloat16),
                 preferred_element_type=jnp.float32).astype(jnp.bfloat16)
    lo = lax.bitcast_convert_type(xb[:, :128], jnp.uint16).astype(jnp.int32)
    hi = lax.bitcast_convert_type(xb[:, 128:], jnp.uint16).astype(jnp.int32)
    o_ref[0] = (hi << 16) | lo


def _compute_xw(x, weight):
    BN = 400
    return pl.pallas_call(
        _xw_body,
        grid=(N_NODES // BN, NUM_RELS),
        in_specs=[
            pl.BlockSpec((BN, IN_FEAT), lambda n, r: (n, 0)),
            pl.BlockSpec((1, IN_FEAT, OUT_FEAT), lambda n, r: (r, 0, 0)),
        ],
        out_specs=pl.BlockSpec((1, BN, 128), lambda n, r: (r, n, 0)),
        out_shape=jax.ShapeDtypeStruct((NUM_RELS, N_NODES, 128), jnp.int32),
    )(x, weight)


def _sc_partition(src, etypes, dst, norm1d):
    mesh = plsc.VectorSubcoreMesh(core_axis_name="c", subcore_axis_name="s")

    @functools.partial(
        pl.kernel,
        out_type=(
            jax.ShapeDtypeStruct((NWK * CAP,), jnp.int32),
            jax.ShapeDtypeStruct((NWK * CAP,), jnp.int32),
            jax.ShapeDtypeStruct((NWK * CAP,), jnp.int32),
            jax.ShapeDtypeStruct((NWK * LANES,), jnp.int32),
        ),
        mesh=mesh,
        scratch_types=[
            pltpu.VMEM((ABLK,), jnp.int32),
            pltpu.VMEM((ABLK,), jnp.int32),
            pltpu.VMEM((ABLK,), jnp.int32),
            pltpu.VMEM((ABLK,), jnp.float32),
            pltpu.VMEM((ABLK,), jnp.int32),
            pltpu.VMEM((ABLK,), jnp.int32),
            pltpu.VMEM((ABLK,), jnp.int32),
            pltpu.VMEM((ABLK,), jnp.float32),
            pltpu.VMEM((CAP,), jnp.int32),
            pltpu.VMEM((CAP,), jnp.int32),
            pltpu.VMEM((CAP,), jnp.int32),
            pltpu.VMEM((LANES,), jnp.int32),
            pltpu.SemaphoreType.DMA,
            pltpu.SemaphoreType.DMA,
        ],
        compiler_params=_sc_compiler_params(),
    )
    def a_kernel(src_hbm, et_hbm, dst_hbm, norm_hbm,
                 ridx_out, norm_out, ldst_out, cnt_out,
                 src_a, et_a, dst_a, nrm_a, src_b, et_b, dst_b, nrm_b,
                 ridx_l, norm_l, ldst_l, cnt_v, sem_a, sem_b):
        c = lax.axis_index("c")
        s = lax.axis_index("s")
        w = s * NC + c
        d0 = w * BR
        NB = N_EDGES // ABLK

        def issue(b, bufs, sem):
            e0 = b * ABLK
            sv, ev, dv, nv = bufs
            pltpu.async_copy(src_hbm.at[pl.ds(e0, ABLK)], sv, sem)
            pltpu.async_copy(et_hbm.at[pl.ds(e0, ABLK)], ev, sem)
            pltpu.async_copy(dst_hbm.at[pl.ds(e0, ABLK)], dv, sem)
            pltpu.async_copy(norm_hbm.at[pl.ds(e0, ABLK)], nv, sem)

        def drain(b, bufs, sem):
            e0 = b * ABLK
            sv, ev, dv, nv = bufs
            pltpu.make_async_copy(src_hbm.at[pl.ds(e0, ABLK)], sv, sem).wait()
            pltpu.make_async_copy(et_hbm.at[pl.ds(e0, ABLK)], ev, sem).wait()
            pltpu.make_async_copy(dst_hbm.at[pl.ds(e0, ABLK)], dv, sem).wait()
            pltpu.make_async_copy(norm_hbm.at[pl.ds(e0, ABLK)], nv, sem).wait()

        def scan(bufs, offv0):
            sv, ev, dv, nv = bufs

            def chunk_body(q, offv):
                sl = pl.ds(q * LANES, LANES)
                ld = dv[sl] - d0
                m = (ld >= 0) & (ld < BR)
                pos = offv + plsc.cumsum(jnp.where(m, 1, 0)) - 1
                ridx = ev[sl] * N_NODES + sv[sl]
                nb = (lax.bitcast_convert_type(nv[sl], jnp.int32)
                      + 0x8000) >> 16
                nw = (nb << 16) | nb
                plsc.store_scatter(ridx_l, [pos], ridx, mask=m)
                plsc.store_scatter(norm_l, [pos], nw, mask=m)
                plsc.store_scatter(ldst_l, [pos], ld, mask=m)
                return offv + plsc.all_reduce_population_count(m)

            return lax.fori_loop(0, ABLK // LANES, chunk_body, offv0,
                                 unroll=8)

        bufs_a = (src_a, et_a, dst_a, nrm_a)
        bufs_b = (src_b, et_b, dst_b, nrm_b)
        issue(0, bufs_a, sem_a)

        def pair_body(p, offv):
            b0 = 2 * p
            issue(b0 + 1, bufs_b, sem_b)
            drain(b0, bufs_a, sem_a)
            offv = scan(bufs_a, offv)

            @pl.when(b0 + 2 < NB)
            def _():
                issue(b0 + 2, bufs_a, sem_a)

            drain(b0 + 1, bufs_b, sem_b)
            return scan(bufs_b, offv)

        offv = lax.fori_loop(0, NB // 2, pair_body,
                             jnp.zeros((LANES,), jnp.int32))
        off = offv[0]

        pad_ridx = jnp.full((LANES,), w * 128, jnp.int32)
        pad_norm = jnp.zeros((LANES,), jnp.int32)
        pad_ldst = jnp.full((LANES,), BR, jnp.int32)
        full = pad_ldst >= 0
        for i in range(CHUNK // LANES):
            o = off + i * LANES
            plsc.store_compressed(ridx_l.at[pl.ds(o, LANES)], pad_ridx, mask=full)
            plsc.store_compressed(norm_l.at[pl.ds(o, LANES)], pad_norm, mask=full)
            plsc.store_compressed(ldst_l.at[pl.ds(o, LANES)], pad_ldst, mask=full)
        count_p = (off // CHUNK + 1) * CHUNK

        cnt_v[...] = jnp.full((LANES,), 0, jnp.int32) + count_p
        pltpu.sync_copy(cnt_v, cnt_out.at[pl.ds(w * LANES, LANES)])
        pltpu.sync_copy(ridx_l, ridx_out.at[pl.ds(w * CAP, CAP)])
        pltpu.sync_copy(norm_l, norm_out.at[pl.ds(w * CAP, CAP)])
        pltpu.sync_copy(ldst_l, ldst_out.at[pl.ds(w * CAP, CAP)])

    return a_kernel(src, etypes, dst, norm1d)


def _sc_gather_acc(xw_flat, ridx, nrm, ldst, cnts):
    mesh = plsc.VectorSubcoreMesh(core_axis_name="c", subcore_axis_name="s")

    @functools.partial(
        pl.kernel,
        out_type=jax.ShapeDtypeStruct((N_NODES, 128), jnp.int32),
        mesh=mesh,
        scratch_types=[
            pltpu.VMEM((CAP,), jnp.int32),
            pltpu.VMEM((CAP + LANES,), jnp.int32),
            pltpu.VMEM((CAP + LANES,), jnp.int32),
            pltpu.VMEM((CHUNK, 128), jnp.int32),
            pltpu.VMEM((CHUNK, 128), jnp.int32),
            pltpu.VMEM((ACC_R, 128), jnp.int32),
            pltpu.VMEM((LANES,), jnp.int32),
            pltpu.SemaphoreType.DMA,
            pltpu.SemaphoreType.DMA,
            pltpu.SemaphoreType.DMA,
        ],
        compiler_params=_sc_compiler_params(),
    )
    def b_kernel(xw_hbm, ridx_hbm, norm_hbm, ldst_hbm, cnt_hbm,
                 out_hbm, idx_l, norm_l, ldst_l, rows_a, rows_b, acc_v,
                 cnt_v, sem_l, sem_a, sem_b):
        c = lax.axis_index("c")
        s = lax.axis_index("s")
        w = s * NC + c
        base = w * CAP

        pltpu.async_copy(ridx_hbm.at[pl.ds(base, CAP)], idx_l, sem_l)
        pltpu.async_copy(norm_hbm.at[pl.ds(base, CAP)],
                         norm_l.at[pl.ds(0, CAP)], sem_l)
        pltpu.async_copy(ldst_hbm.at[pl.ds(base, CAP)],
                         ldst_l.at[pl.ds(0, CAP)], sem_l)

        zv = jnp.zeros((LANES,), jnp.int32)

        @pl.loop(0, ACC_R)
        def _(r):
            for l in range(8):
                acc_v.at[r][pl.ds(l * LANES, LANES)] = zv

        pltpu.sync_copy(cnt_hbm.at[pl.ds(w * LANES, LANES)], cnt_v)
        nc = cnt_v[...][0] // CHUNK

        pltpu.make_async_copy(ridx_hbm.at[pl.ds(base, CAP)],
                              idx_l, sem_l).wait()
        pltpu.make_async_copy(norm_hbm.at[pl.ds(base, CAP)],
                              norm_l.at[pl.ds(0, CAP)], sem_l).wait()
        pltpu.make_async_copy(ldst_hbm.at[pl.ds(base, CAP)],
                              ldst_l.at[pl.ds(0, CAP)], sem_l).wait()

        def fire(k, rows_v, sem):
            idx_s = idx_l.at[pl.ds(k * CHUNK, CHUNK)]
            pltpu.async_copy(xw_hbm.at[idx_s], rows_v, sem)

        def accumulate(k, rows_v, sem):
            idx_s = idx_l.at[pl.ds(k * CHUNK, CHUNK)]
            pltpu.make_async_copy(xw_hbm.at[idx_s], rows_v, sem).wait()
            o0 = k * CHUNK

            zero16 = jnp.zeros((LANES,), jnp.int32)

            def edge_body(j, carry):
                njw = jnp.take(norm_l[pl.ds(o0 + j, LANES)], zero16,
                               axis=0)
                njv = plsc.bitcast(njw, jnp.bfloat16)
                dj = ldst_l[pl.ds(o0 + j, LANES)][0]
                for l in range(8):
                    sl2 = pl.ds(l * LANES, LANES)
                    prod = plsc.bitcast(rows_v.at[j][sl2],
                                        jnp.bfloat16) * njv
                    cur = plsc.bitcast(acc_v.at[dj][sl2], jnp.bfloat16)
                    acc_v.at[dj][sl2] = plsc.bitcast(cur + prod, jnp.int32)
                return carry

            lax.fori_loop(0, CHUNK, edge_body, 0, unroll=4)

        fire(0, rows_a, sem_a)

        @pl.loop(0, nc, step=2)
        def _(k0):
            @pl.when(k0 + 1 < nc)
            def _():
                fire(k0 + 1, rows_b, sem_b)

            accumulate(k0, rows_a, sem_a)

            @pl.when(k0 + 1 < nc)
            def _():
                @pl.when(k0 + 2 < nc)
                def _():
                    fire(k0 + 2, rows_a, sem_a)

                accumulate(k0 + 1, rows_b, sem_b)

        @pl.when(w < NWK - 1)
        def _():
            pltpu.sync_copy(acc_v.at[pl.ds(0, BR)],
                            out_hbm.at[pl.ds(w * BR, BR)])

        @pl.when(w == NWK - 1)
        def _():
            tail = N_NODES - (NWK - 1) * BR
            pltpu.sync_copy(acc_v.at[pl.ds(0, tail)],
                            out_hbm.at[pl.ds((NWK - 1) * BR, tail)])

    return b_kernel(xw_flat, ridx, nrm, ldst, cnts)


def _bias_body(p_ref, b_ref, o_ref):
    u = p_ref[...]
    lo = lax.bitcast_convert_type((u & 0xFFFF).astype(jnp.uint16),
                                  jnp.bfloat16).astype(jnp.float32)
    hi = lax.bitcast_convert_type((u >> 16).astype(jnp.uint16),
                                  jnp.bfloat16).astype(jnp.float32)
    o_ref[:, :128] = lo + b_ref[pl.ds(0, 128)][None, :]
    o_ref[:, 128:] = hi + b_ref[pl.ds(128, 128)][None, :]


def _tc_bias(partial, h_bias):
    BN = 2000
    return pl.pallas_call(
        _bias_body,
        grid=(N_NODES // BN,),
        in_specs=[
            pl.BlockSpec((BN, 128), lambda n: (n, 0)),
            pl.BlockSpec((OUT_FEAT,), lambda n: (0,)),
        ],
        out_specs=pl.BlockSpec((BN, OUT_FEAT), lambda n: (n, 0)),
        out_shape=jax.ShapeDtypeStruct((N_NODES, OUT_FEAT), jnp.float32),
        compiler_params=pltpu.CompilerParams(
            dimension_semantics=("parallel",),
        ),
    )(partial, h_bias)


def kernel(x, edge_index, etypes, norm, weight, h_bias):
    xw = _compute_xw(x, weight)
    xw_flat = xw.reshape(NUM_RELS * N_NODES, 128)
    src = edge_index[0]
    dst = edge_index[1]
    ridx, nrm, ldst, cnts = _sc_partition(src, etypes, dst, norm.reshape(-1))
    acc = _sc_gather_acc(xw_flat, ridx, nrm, ldst, cnts)
    return _tc_bias(acc, h_bias)
